# baseline (speedup 1.0000x reference)
import jax
import jax.numpy as jnp
from jax.experimental import pallas as pl

N = 10000
E = 160000
G = 64
L = 3


def _mlp_apply(h, layers):
    n = len(layers)
    for i, (W, b) in enumerate(layers):
        h = h @ W + b
        if i < n - 1:
            h = jax.nn.relu(h)
    return h


def _add_kernel(a_ref, b_ref, o_ref):
    o_ref[...] = a_ref[...] + b_ref[...]


def _padd(a, b):
    return pl.pallas_call(
        _add_kernel,
        out_shape=jax.ShapeDtypeStruct(a.shape, a.dtype),
    )(a, b)


def kernel(x, edge_index, edge_attr, batch, params):
    src = edge_index[0]
    dst = edge_index[1]
    counts = jnp.bincount(batch, length=G)
    scale = 1.0 / jnp.sqrt(jnp.maximum(counts, 1).astype(jnp.float32))
    node_scale = scale[batch][:, None]
    reps = [x]
    h = x
    for i in range(L):
        cp = params['convs'][i]
        Wl, bl = cp['lin_edge']
        eemb = edge_attr @ Wl + bl
        msg = jax.nn.relu(h[src] + eemb)
        agg = jax.ops.segment_sum(msg, dst, num_segments=N)
        out = jax.nn.relu(_mlp_apply((1.0 + cp['eps']) * h + agg, cp['mlp']))
        h = out + h if i > 0 else out
        reps.append(h)
    sum_pool = None
    for i in range(L + 1):
        pooled = jax.ops.segment_sum(reps[i] * node_scale, batch, num_segments=G)
        z = _mlp_apply(pooled, params['readouts'][i])
        sum_pool = z if sum_pool is None else _padd(sum_pool, z)
    return sum_pool


# speedup vs baseline: 1.9341x; 1.9341x over previous
import functools

import jax
import jax.numpy as jnp
from jax import lax
from jax.experimental import pallas as pl
from jax.experimental.pallas import tpu as pltpu
from jax.experimental.pallas import tpu_sc as plsc

N = 10000
E = 160000
DIN = 256
DH = 256
DOUT = 128
L = 3
G = 64
HALF = 128

NSUB = 16
EPT = E // NSUB
CB = 80
NCHUNK = EPT // CB
NPAD = 10240
ROWS_PER_TILE = NPAD // NSUB


def _edge_body(h2_hbm, srcs_hbm, dst_hbm, a0_hbm, a1_hbm, w_hbm, zin_hbm,
               out_hbm, wv, sbuf, isrc, idst, a0b, a1b, aggsh):
    c = lax.axis_index("c")
    s = lax.axis_index("s")
    base = s * EPT

    pltpu.sync_copy(w_hbm.at[c], wv)
    pltpu.sync_copy(zin_hbm, aggsh.at[pl.ds(s * ROWS_PER_TILE, ROWS_PER_TILE)])
    plsc.subcore_barrier()

    w0 = [wv[0, pl.ds(16 * j, 16)] for j in range(8)]
    w1 = [wv[1, pl.ds(16 * j, 16)] for j in range(8)]
    wb = [wv[2, pl.ds(16 * j, 16)] for j in range(8)]

    @pl.loop(0, NCHUNK)
    def _(k):
        off = base + k * CB
        pltpu.sync_copy(srcs_hbm.at[pl.ds(c * E + off, CB)], isrc.at[0])
        pltpu.sync_copy(dst_hbm.at[pl.ds(off, CB)], idst.at[0])
        pltpu.sync_copy(a0_hbm.at[pl.ds(off, CB)], a0b.at[0])
        pltpu.sync_copy(a1_hbm.at[pl.ds(off, CB)], a1b.at[0])

        pltpu.sync_copy(h2_hbm.at[isrc.at[0]], sbuf)

        @pl.loop(0, CB // 16)
        def _(e16):
            eb = e16 * 16
            a0v = a0b[0, pl.ds(eb, 16)]
            a1v = a1b[0, pl.ds(eb, 16)]
            for t in range(16):
                a0s = a0v[t]
                a1s = a1v[t]
                for j in range(8):
                    sl = pl.ds(16 * j, 16)
                    v = sbuf[eb + t, sl]
                    v = jnp.maximum(v + a0s * w0[j] + a1s * w1[j] + wb[j], 0.0)
                    sbuf[eb + t, sl] = v

        pltpu.sync_copy(sbuf, aggsh.at[idst.at[0]], add=True)

    plsc.subcore_barrier()
    pltpu.sync_copy(aggsh.at[pl.ds(s * ROWS_PER_TILE, ROWS_PER_TILE)],
                    out_hbm.at[c, pl.ds(s * ROWS_PER_TILE, ROWS_PER_TILE)])


@jax.jit
def _edge_agg(h2flat, srcs, dst, a0, a1, wconst, zin):
    mesh = plsc.VectorSubcoreMesh(core_axis_name="c", subcore_axis_name="s")
    return pl.kernel(
        _edge_body,
        out_type=jax.ShapeDtypeStruct((2, NPAD, HALF), jnp.float32),
        mesh=mesh,
        scratch_types=[
            pltpu.VMEM((3, HALF), jnp.float32),
            pltpu.VMEM((CB, HALF), jnp.float32),
            pltpu.VMEM((1, CB), jnp.int32),
            pltpu.VMEM((1, CB), jnp.int32),
            pltpu.VMEM((1, CB), jnp.float32),
            pltpu.VMEM((1, CB), jnp.float32),
            pltpu.VMEM_SHARED((NPAD, HALF), jnp.float32),
        ],
    )(h2flat, srcs, dst, a0, a1, wconst, zin)


BR = 2000


def _layer_kernel(first, h_ref, a_ref, eps_ref, w1_ref, b1_ref, w2_ref,
                  b2_ref, w3_ref, b3_ref, out_ref):
    h = jnp.concatenate([h_ref[0], h_ref[1]], axis=1)
    agg = jnp.concatenate([a_ref[0], a_ref[1]], axis=1)
    u = (1.0 + eps_ref[0, 0]) * h + agg
    t = jnp.maximum(jnp.dot(u, w1_ref[...],
                            preferred_element_type=jnp.float32) + b1_ref[...], 0.0)
    t = jnp.maximum(jnp.dot(t, w2_ref[...],
                            preferred_element_type=jnp.float32) + b2_ref[...], 0.0)
    t = jnp.dot(t, w3_ref[...], preferred_element_type=jnp.float32) + b3_ref[...]
    t = jnp.maximum(t, 0.0)
    if not first:
        t = t + h
    out_ref[0] = t[:, :HALF]
    out_ref[1] = t[:, HALF:]


@functools.partial(jax.jit, static_argnums=(2,))
def _layer_tc(h2, agg2, first, eps, w1, b1, w2, b2, w3, b3):
    grid = (N // BR,)
    bs_w = pl.BlockSpec((DH, DH), lambda i: (0, 0))
    bs_b = pl.BlockSpec((1, DH), lambda i: (0, 0))
    return pl.pallas_call(
        functools.partial(_layer_kernel, first),
        grid=grid,
        in_specs=[
            pl.BlockSpec((2, BR, HALF), lambda i: (0, i, 0)),
            pl.BlockSpec((2, BR, HALF), lambda i: (0, i, 0)),
            pl.BlockSpec((1, 1), lambda i: (0, 0)),
            bs_w, bs_b, bs_w, bs_b, bs_w, bs_b,
        ],
        out_specs=pl.BlockSpec((2, BR, HALF), lambda i: (0, i, 0)),
        out_shape=jax.ShapeDtypeStruct((2, N, HALF), jnp.float32),
    )(h2, agg2, eps, w1, b1, w2, b2, w3, b3)


def _finale_kernel(r0_ref, r1_ref, r2_ref, r3_ref, batch_ref, ra_ref, rc_ref,
                   rb_ref, rd_ref, out_ref, pool_acc, cnt_acc):
    i = pl.program_id(0)

    @pl.when(i == 0)
    def _():
        pool_acc[...] = jnp.zeros_like(pool_acc)
        cnt_acc[...] = jnp.zeros_like(cnt_acc)

    bvec = batch_ref[0]
    gids = lax.broadcasted_iota(jnp.int32, (G, BR), 0)
    oht = (gids == jnp.broadcast_to(bvec, (G, BR))).astype(jnp.float32)
    cnt_acc[...] += jnp.dot(oht, jnp.ones((BR, HALF), jnp.float32),
                            preferred_element_type=jnp.float32)
    for r, ref in enumerate((r0_ref, r1_ref, r2_ref, r3_ref)):
        rep = jnp.concatenate([ref[0], ref[1]], axis=1)
        pool_acc[r] += jnp.dot(oht, rep, preferred_element_type=jnp.float32)

    @pl.when(i == pl.num_programs(0) - 1)
    def _():
        scale_h = lax.rsqrt(jnp.maximum(cnt_acc[...], 1.0))
        scale = jnp.concatenate([scale_h, scale_h], axis=1)
        z = jnp.zeros((G, DOUT), jnp.float32)
        for r in range(4):
            p = pool_acc[r] * scale
            t = jnp.maximum(jnp.dot(p, ra_ref[r],
                                    preferred_element_type=jnp.float32)
                            + rc_ref[r], 0.0)
            z = z + jnp.dot(t, rb_ref[r],
                            preferred_element_type=jnp.float32) + rd_ref[r]
        out_ref[...] = z


@jax.jit
def _finale_tc(r0, r1, r2, r3, batch3, ra, rc, rb, rd):
    grid = (N // BR,)
    bs_rep = pl.BlockSpec((2, BR, HALF), lambda i: (0, i, 0))
    return pl.pallas_call(
        _finale_kernel,
        grid=grid,
        in_specs=[
            bs_rep, bs_rep, bs_rep, bs_rep,
            pl.BlockSpec((1, 1, BR), lambda i: (i, 0, 0)),
            pl.BlockSpec((4, DH, DH), lambda i: (0, 0, 0)),
            pl.BlockSpec((4, 1, DH), lambda i: (0, 0, 0)),
            pl.BlockSpec((4, DH, DOUT), lambda i: (0, 0, 0)),
            pl.BlockSpec((4, 1, DOUT), lambda i: (0, 0, 0)),
        ],
        out_specs=pl.BlockSpec((G, DOUT), lambda i: (0, 0)),
        out_shape=jax.ShapeDtypeStruct((G, DOUT), jnp.float32),
        scratch_shapes=[
            pltpu.VMEM((4, G, DH), jnp.float32),
            pltpu.VMEM((G, HALF), jnp.float32),
        ],
    )(r0, r1, r2, r3, batch3, ra, rc, rb, rd)


def kernel(x, edge_index, edge_attr, batch, params):
    src = edge_index[0].astype(jnp.int32)
    dst = edge_index[1].astype(jnp.int32)
    srcs = jnp.concatenate([src, src + N])
    a0 = edge_attr[:, 0]
    a1 = edge_attr[:, 1]
    zin = jnp.zeros((ROWS_PER_TILE, HALF), jnp.float32)
    batch3 = batch.astype(jnp.int32).reshape(N // BR, 1, BR)

    h2 = x.reshape(N, 2, HALF).transpose(1, 0, 2)
    reps = [h2]
    for i in range(L):
        cp = params['convs'][i]
        Wl, bl = cp['lin_edge']
        wconst = jnp.stack([
            jnp.stack([Wl[0, :HALF], Wl[1, :HALF], bl[:HALF]]),
            jnp.stack([Wl[0, HALF:], Wl[1, HALF:], bl[HALF:]]),
        ])
        agg2 = _edge_agg(h2.reshape(2 * N, HALF), srcs, dst, a0, a1, wconst, zin)
        (W1, b1), (W2, b2), (W3, b3) = cp['mlp']
        h2 = _layer_tc(h2, agg2, i == 0, cp['eps'].reshape(1, 1),
                       W1, b1.reshape(1, DH), W2, b2.reshape(1, DH),
                       W3, b3.reshape(1, DH))
        reps.append(h2)

    ra = jnp.stack([params['readouts'][i][0][0] for i in range(4)])
    rc = jnp.stack([params['readouts'][i][0][1].reshape(1, DH) for i in range(4)])
    rb = jnp.stack([params['readouts'][i][1][0] for i in range(4)])
    rd = jnp.stack([params['readouts'][i][1][1].reshape(1, DOUT) for i in range(4)])
    return _finale_tc(reps[0], reps[1], reps[2], reps[3], batch3, ra, rc, rb, rd)


# speedup vs baseline: 4.4707x; 2.3114x over previous
import functools

import jax
import jax.numpy as jnp
from jax import lax
from jax.experimental import pallas as pl
from jax.experimental.pallas import tpu as pltpu
from jax.experimental.pallas import tpu_sc as plsc

N = 10000
E = 160000
DIN = 256
DH = 256
DOUT = 128
L = 3
G = 64
HALF = 128

NSUB = 16
EPT = E // NSUB
SUP = 2000
NSUP = EPT // SUP
CB = 80
NCHUNK = SUP // CB
NBUF = 3
NPAD = 10240
ROWS_PER_TILE = NPAD // NSUB


def _edge_body(h2_hbm, srcs_hbm, dst_hbm, a0_hbm, a1_hbm, w_hbm, zin_hbm,
               out_hbm, wv, sbuf, isup, dsup, a0sup, a1sup, istage, dstage,
               aggsh, gat_sems, scat_sems):
    c = lax.axis_index("c")
    s = lax.axis_index("s")

    pltpu.sync_copy(w_hbm.at[c], wv)
    pltpu.sync_copy(zin_hbm, aggsh.at[pl.ds(s * ROWS_PER_TILE, ROWS_PER_TILE)])
    plsc.subcore_barrier()

    w0 = [wv[0, pl.ds(16 * j, 16)] for j in range(8)]
    w1 = [wv[1, pl.ds(16 * j, 16)] for j in range(8)]
    wb = [wv[2, pl.ds(16 * j, 16)] for j in range(8)]

    def stage(kk, b):
        for i in range(CB // 16):
            sl = pl.ds(kk * CB + 16 * i, 16)
            dl = pl.ds(16 * i, 16)
            istage[b, dl] = isup[0, sl]
            dstage[b, dl] = dsup[0, sl]

    def gat_start(b):
        pltpu.async_copy(h2_hbm.at[istage.at[b]], sbuf.at[b], gat_sems[b])

    def gat_wait(b):
        pltpu.make_async_copy(h2_hbm.at[istage.at[b]], sbuf.at[b],
                              gat_sems[b]).wait()

    def scat_start(b):
        pltpu.async_copy(sbuf.at[b], aggsh.at[dstage.at[b]], scat_sems[b],
                         add=True)

    def scat_wait(b):
        pltpu.make_async_copy(sbuf.at[b], aggsh.at[dstage.at[b]],
                              scat_sems[b]).wait()

    def compute(kk, b):
        @pl.loop(0, CB // 16)
        def _(e16):
            eb = e16 * 16
            a0v = a0sup[0, pl.ds(kk * CB + eb, 16)]
            a1v = a1sup[0, pl.ds(kk * CB + eb, 16)]
            for t in range(16):
                a0s = a0v[t]
                a1s = a1v[t]
                for j in range(8):
                    sl = pl.ds(16 * j, 16)
                    v = sbuf[b, eb + t, sl]
                    v = jnp.maximum(v + a0s * w0[j] + a1s * w1[j] + wb[j], 0.0)
                    sbuf[b, eb + t, sl] = v

    @pl.loop(0, NSUP)
    def _(sup):
        soff = s * EPT + sup * SUP
        pltpu.sync_copy(srcs_hbm.at[pl.ds(c * E + soff, SUP)], isup.at[0])
        pltpu.sync_copy(dst_hbm.at[pl.ds(soff, SUP)], dsup.at[0])
        pltpu.sync_copy(a0_hbm.at[pl.ds(soff, SUP)], a0sup.at[0])
        pltpu.sync_copy(a1_hbm.at[pl.ds(soff, SUP)], a1sup.at[0])

        for q in range(NBUF):
            stage(q, q)
            gat_start(q)

        @pl.loop(0, NCHUNK - 1, step=NBUF)
        def _(k):
            for q in range(NBUF):
                kk = k + q
                b = q
                gat_wait(b)
                compute(kk, b)
                scat_start(b)
                nb = (q + 2) % NBUF

                @pl.when(jnp.logical_and(kk + 2 >= NBUF, kk + 2 < NCHUNK))
                def _():
                    scat_wait(nb)
                    stage(kk + 2, nb)
                    gat_start(nb)

        bt = (NCHUNK - 1) % NBUF
        gat_wait(bt)
        compute(NCHUNK - 1, bt)
        scat_start(bt)
        for q in range(NBUF):
            scat_wait(q)

    plsc.subcore_barrier()
    pltpu.sync_copy(aggsh.at[pl.ds(s * ROWS_PER_TILE, ROWS_PER_TILE)],
                    out_hbm.at[c, pl.ds(s * ROWS_PER_TILE, ROWS_PER_TILE)])


@jax.jit
def _edge_agg(h2flat, srcs, dst, a0, a1, wconst, zin):
    mesh = plsc.VectorSubcoreMesh(core_axis_name="c", subcore_axis_name="s")
    return pl.kernel(
        _edge_body,
        out_type=jax.ShapeDtypeStruct((2, NPAD, HALF), jnp.float32),
        mesh=mesh,
        scratch_types=[
            pltpu.VMEM((3, HALF), jnp.float32),
            pltpu.VMEM((NBUF, CB, HALF), jnp.float32),
            pltpu.VMEM((1, SUP), jnp.int32),
            pltpu.VMEM((1, SUP), jnp.int32),
            pltpu.VMEM((1, SUP), jnp.float32),
            pltpu.VMEM((1, SUP), jnp.float32),
            pltpu.VMEM((NBUF, CB), jnp.int32),
            pltpu.VMEM((NBUF, CB), jnp.int32),
            pltpu.VMEM_SHARED((NPAD, HALF), jnp.float32),
            [pltpu.SemaphoreType.DMA] * NBUF,
            [pltpu.SemaphoreType.DMA] * NBUF,
        ],
    )(h2flat, srcs, dst, a0, a1, wconst, zin)


BR = 2000


def _layer_kernel(first, h_ref, a_ref, eps_ref, w1_ref, b1_ref, w2_ref,
                  b2_ref, w3_ref, b3_ref, out_ref):
    h = jnp.concatenate([h_ref[0], h_ref[1]], axis=1)
    agg = jnp.concatenate([a_ref[0], a_ref[1]], axis=1)
    u = (1.0 + eps_ref[0, 0]) * h + agg
    t = jnp.maximum(jnp.dot(u, w1_ref[...],
                            preferred_element_type=jnp.float32) + b1_ref[...], 0.0)
    t = jnp.maximum(jnp.dot(t, w2_ref[...],
                            preferred_element_type=jnp.float32) + b2_ref[...], 0.0)
    t = jnp.dot(t, w3_ref[...], preferred_element_type=jnp.float32) + b3_ref[...]
    t = jnp.maximum(t, 0.0)
    if not first:
        t = t + h
    out_ref[0] = t[:, :HALF]
    out_ref[1] = t[:, HALF:]


@functools.partial(jax.jit, static_argnums=(2,))
def _layer_tc(h2, agg2, first, eps, w1, b1, w2, b2, w3, b3):
    grid = (N // BR,)
    bs_w = pl.BlockSpec((DH, DH), lambda i: (0, 0))
    bs_b = pl.BlockSpec((1, DH), lambda i: (0, 0))
    return pl.pallas_call(
        functools.partial(_layer_kernel, first),
        grid=grid,
        in_specs=[
            pl.BlockSpec((2, BR, HALF), lambda i: (0, i, 0)),
            pl.BlockSpec((2, BR, HALF), lambda i: (0, i, 0)),
            pl.BlockSpec((1, 1), lambda i: (0, 0)),
            bs_w, bs_b, bs_w, bs_b, bs_w, bs_b,
        ],
        out_specs=pl.BlockSpec((2, BR, HALF), lambda i: (0, i, 0)),
        out_shape=jax.ShapeDtypeStruct((2, N, HALF), jnp.float32),
    )(h2, agg2, eps, w1, b1, w2, b2, w3, b3)


def _finale_kernel(r0_ref, r1_ref, r2_ref, r3_ref, batch_ref, ra_ref, rc_ref,
                   rb_ref, rd_ref, out_ref, pool_acc, cnt_acc):
    i = pl.program_id(0)

    @pl.when(i == 0)
    def _():
        pool_acc[...] = jnp.zeros_like(pool_acc)
        cnt_acc[...] = jnp.zeros_like(cnt_acc)

    bvec = batch_ref[0]
    gids = lax.broadcasted_iota(jnp.int32, (G, BR), 0)
    oht = (gids == jnp.broadcast_to(bvec, (G, BR))).astype(jnp.float32)
    cnt_acc[...] += jnp.dot(oht, jnp.ones((BR, HALF), jnp.float32),
                            preferred_element_type=jnp.float32)
    for r, ref in enumerate((r0_ref, r1_ref, r2_ref, r3_ref)):
        rep = jnp.concatenate([ref[0], ref[1]], axis=1)
        pool_acc[r] += jnp.dot(oht, rep, preferred_element_type=jnp.float32)

    @pl.when(i == pl.num_programs(0) - 1)
    def _():
        scale_h = lax.rsqrt(jnp.maximum(cnt_acc[...], 1.0))
        scale = jnp.concatenate([scale_h, scale_h], axis=1)
        z = jnp.zeros((G, DOUT), jnp.float32)
        for r in range(4):
            p = pool_acc[r] * scale
            t = jnp.maximum(jnp.dot(p, ra_ref[r],
                                    preferred_element_type=jnp.float32)
                            + rc_ref[r], 0.0)
            z = z + jnp.dot(t, rb_ref[r],
                            preferred_element_type=jnp.float32) + rd_ref[r]
        out_ref[...] = z


@jax.jit
def _finale_tc(r0, r1, r2, r3, batch3, ra, rc, rb, rd):
    grid = (N // BR,)
    bs_rep = pl.BlockSpec((2, BR, HALF), lambda i: (0, i, 0))
    return pl.pallas_call(
        _finale_kernel,
        grid=grid,
        in_specs=[
            bs_rep, bs_rep, bs_rep, bs_rep,
            pl.BlockSpec((1, 1, BR), lambda i: (i, 0, 0)),
            pl.BlockSpec((4, DH, DH), lambda i: (0, 0, 0)),
            pl.BlockSpec((4, 1, DH), lambda i: (0, 0, 0)),
            pl.BlockSpec((4, DH, DOUT), lambda i: (0, 0, 0)),
            pl.BlockSpec((4, 1, DOUT), lambda i: (0, 0, 0)),
        ],
        out_specs=pl.BlockSpec((G, DOUT), lambda i: (0, 0)),
        out_shape=jax.ShapeDtypeStruct((G, DOUT), jnp.float32),
        scratch_shapes=[
            pltpu.VMEM((4, G, DH), jnp.float32),
            pltpu.VMEM((G, HALF), jnp.float32),
        ],
    )(r0, r1, r2, r3, batch3, ra, rc, rb, rd)


def kernel(x, edge_index, edge_attr, batch, params):
    src = edge_index[0].astype(jnp.int32)
    dst = edge_index[1].astype(jnp.int32)
    srcs = jnp.concatenate([src, src + N])
    a0 = edge_attr[:, 0]
    a1 = edge_attr[:, 1]
    zin = jnp.zeros((ROWS_PER_TILE, HALF), jnp.float32)
    batch3 = batch.astype(jnp.int32).reshape(N // BR, 1, BR)

    h2 = x.reshape(N, 2, HALF).transpose(1, 0, 2)
    reps = [h2]
    for i in range(L):
        cp = params['convs'][i]
        Wl, bl = cp['lin_edge']
        wconst = jnp.stack([
            jnp.stack([Wl[0, :HALF], Wl[1, :HALF], bl[:HALF]]),
            jnp.stack([Wl[0, HALF:], Wl[1, HALF:], bl[HALF:]]),
        ])
        agg2 = _edge_agg(h2.reshape(2 * N, HALF), srcs, dst, a0, a1, wconst, zin)
        (W1, b1), (W2, b2), (W3, b3) = cp['mlp']
        h2 = _layer_tc(h2, agg2, i == 0, cp['eps'].reshape(1, 1),
                       W1, b1.reshape(1, DH), W2, b2.reshape(1, DH),
                       W3, b3.reshape(1, DH))
        reps.append(h2)

    ra = jnp.stack([params['readouts'][i][0][0] for i in range(4)])
    rc = jnp.stack([params['readouts'][i][0][1].reshape(1, DH) for i in range(4)])
    rb = jnp.stack([params['readouts'][i][1][0] for i in range(4)])
    rd = jnp.stack([params['readouts'][i][1][1].reshape(1, DOUT) for i in range(4)])
    return _finale_tc(reps[0], reps[1], reps[2], reps[3], batch3, ra, rc, rb, rd)
